# baseline (speedup 1.0000x reference)
import jax
import jax.numpy as jnp
from jax import lax
from jax.experimental import pallas as pl
from jax.experimental.pallas import tpu as pltpu
from jax.experimental.pallas import tpu_sc as plsc

EMBED = 64
NW = 32
ROWS_PER_DMA = 128
DMAS_PER_GROUP = 5
GROUP_ROWS = ROWS_PER_DMA * DMAS_PER_GROUP


def _sc_gather(table, idx3, n_rows):
    chunks = idx3.shape[1]
    groups = chunks // DMAS_PER_GROUP
    rows_per_w = chunks * ROWS_PER_DMA

    mesh = plsc.VectorSubcoreMesh(core_axis_name="c", subcore_axis_name="s",
                                  num_cores=2, num_subcores=16)

    def body(table_hbm, idx_hbm, out_hbm, idx_v, buf_v, gsem):
        cid = lax.axis_index("c")
        sid = lax.axis_index("s")
        wid = sid * 2 + cid
        base = wid * rows_per_w
        pltpu.sync_copy(idx_hbm.at[wid], idx_v)

        def group(g, _):
            cps = []
            for k in range(DMAS_PER_GROUP):
                j = g * DMAS_PER_GROUP + k
                cps.append(pltpu.async_copy(
                    table_hbm.at[idx_v.at[j]],
                    buf_v.at[pl.ds(k * ROWS_PER_DMA, ROWS_PER_DMA)],
                    gsem))
            for cp in cps:
                cp.wait()
            pltpu.sync_copy(buf_v, out_hbm.at[pl.ds(base + g * GROUP_ROWS,
                                                    GROUP_ROWS)])
            return 0

        lax.fori_loop(0, groups, group, 0)

    gathered = pl.kernel(
        body,
        out_type=jax.ShapeDtypeStruct((n_rows, EMBED), jnp.float32),
        mesh=mesh,
        compiler_params=pltpu.CompilerParams(use_tc_tiling_on_sc=False),
        scratch_types=[
            pltpu.VMEM((chunks, ROWS_PER_DMA), jnp.int32),
            pltpu.VMEM((GROUP_ROWS, EMBED), jnp.float32),
            pltpu.SemaphoreType.DMA,
        ],
    )(table, idx3)
    return gathered


def _tc_linear(x, w, b2):
    n = x.shape[0]
    bm = 2048
    grid = n // bm

    def body(x_ref, w_ref, b_ref, o_ref):
        o_ref[...] = jnp.dot(x_ref[...], w_ref[...],
                             preferred_element_type=jnp.float32) + b_ref[...]

    return pl.pallas_call(
        body,
        grid=(grid,),
        in_specs=[
            pl.BlockSpec((bm, EMBED), lambda i: (i, 0)),
            pl.BlockSpec((EMBED, EMBED), lambda i: (0, 0)),
            pl.BlockSpec((1, EMBED), lambda i: (0, 0)),
        ],
        out_specs=pl.BlockSpec((bm, EMBED), lambda i: (i, 0)),
        out_shape=jax.ShapeDtypeStruct((n, EMBED), jnp.float32),
    )(x, w, b2)


def kernel(qus, table, W, b):
    batch, seq = qus.shape
    n = batch * seq
    idx3 = qus.reshape(NW, n // (NW * ROWS_PER_DMA), ROWS_PER_DMA)
    idx3 = idx3.astype(jnp.int32)
    gathered = _sc_gather(table, idx3, n)
    out = _tc_linear(gathered, W, b.reshape(1, EMBED))
    return out.reshape(batch, seq, EMBED)


# speedup vs baseline: 1.3288x; 1.3288x over previous
import jax
import jax.numpy as jnp
from jax import lax
from jax.experimental import pallas as pl
from jax.experimental.pallas import tpu as pltpu
from jax.experimental.pallas import tpu_sc as plsc

EMBED = 64
NW = 32
ROWS_PER_DMA = 128
DMAS_PER_GROUP = 5
GROUP_ROWS = ROWS_PER_DMA * DMAS_PER_GROUP
OUT_W = 128


def _sc_gather(table, idx3, n_rows):
    chunks = idx3.shape[1]
    groups = chunks // DMAS_PER_GROUP
    rows_per_w = chunks * ROWS_PER_DMA

    mesh = plsc.VectorSubcoreMesh(core_axis_name="c", subcore_axis_name="s",
                                  num_cores=2, num_subcores=16)

    def body(table_hbm, idx_hbm, out_hbm, idx_v, buf0, buf1, gsem,
             wsem0, wsem1):
        bufs = (buf0, buf1)
        wsems = (wsem0, wsem1)
        cid = lax.axis_index("c")
        sid = lax.axis_index("s")
        wid = sid * 2 + cid
        base = wid * rows_per_w
        pltpu.sync_copy(idx_hbm.at[wid], idx_v)

        def wb_drain(b):
            pltpu.make_async_copy(
                bufs[b],
                out_hbm.at[pl.ds(0, GROUP_ROWS), pl.ds(0, EMBED)],
                wsems[b]).wait()

        @pl.loop(0, groups, step=2)
        def _(big_g):
            for b in range(2):
                g = big_g + b

                @pl.when(g >= 2)
                def _():
                    wb_drain(b)

                cps = []
                for k in range(DMAS_PER_GROUP):
                    j = g * DMAS_PER_GROUP + k
                    cps.append(pltpu.async_copy(
                        table_hbm.at[idx_v.at[j]],
                        bufs[b].at[pl.ds(k * ROWS_PER_DMA, ROWS_PER_DMA)],
                        gsem))
                for cp in cps:
                    cp.wait()
                pltpu.async_copy(
                    bufs[b],
                    out_hbm.at[pl.ds(base + g * GROUP_ROWS, GROUP_ROWS),
                               pl.ds(0, EMBED)],
                    wsems[b])

        for b in range(2):
            wb_drain(b)

    gathered = pl.kernel(
        body,
        out_type=jax.ShapeDtypeStruct((n_rows, OUT_W), jnp.float32),
        mesh=mesh,
        compiler_params=pltpu.CompilerParams(use_tc_tiling_on_sc=False),
        scratch_types=[
            pltpu.VMEM((chunks, ROWS_PER_DMA), jnp.int32),
            pltpu.VMEM((GROUP_ROWS, EMBED), jnp.float32),
            pltpu.VMEM((GROUP_ROWS, EMBED), jnp.float32),
            pltpu.SemaphoreType.DMA,
            pltpu.SemaphoreType.DMA,
            pltpu.SemaphoreType.DMA,
        ],
    )(table, idx3)
    return gathered


def _tc_linear_t(g3, w, b2, seq, batch):
    bb = 2048
    grid = (seq, batch // bb)

    def body(x_ref, w_ref, b_ref, o_ref):
        x = x_ref[0, :, 0:EMBED]
        o = lax.dot_general(w_ref[...], x, (((0,), (1,)), ((), ())),
                            preferred_element_type=jnp.float32)
        o_ref[0] = o + b_ref[...]

    return pl.pallas_call(
        body,
        grid=grid,
        in_specs=[
            pl.BlockSpec((1, bb, OUT_W), lambda s, c: (s, c, 0)),
            pl.BlockSpec((EMBED, EMBED), lambda s, c: (0, 0)),
            pl.BlockSpec((EMBED, 1), lambda s, c: (0, 0)),
        ],
        out_specs=pl.BlockSpec((1, EMBED, bb), lambda s, c: (s, 0, c)),
        out_shape=jax.ShapeDtypeStruct((seq, EMBED, batch), jnp.float32),
    )(g3, w, b2)


def kernel(qus, table, W, b):
    batch, seq = qus.shape
    n = batch * seq
    idx3 = jnp.swapaxes(qus, 0, 1).reshape(NW, n // (NW * ROWS_PER_DMA),
                                           ROWS_PER_DMA).astype(jnp.int32)
    gathered = _sc_gather(table, idx3, n)
    g3 = gathered.reshape(seq, batch, OUT_W)
    out_t = _tc_linear_t(g3, W, b.reshape(EMBED, 1), seq, batch)
    return jnp.transpose(out_t, (2, 0, 1))


# speedup vs baseline: 2.0263x; 1.5248x over previous
import jax
import jax.numpy as jnp
from jax import lax
from jax.experimental import pallas as pl
from jax.experimental.pallas import tpu as pltpu
from jax.experimental.pallas import tpu_sc as plsc

EMBED = 64
NW = 32
ROWS_PER_DMA = 128
DMAS_PER_GROUP = 5
GROUP_ROWS = ROWS_PER_DMA * DMAS_PER_GROUP
OUT_W = 128
HALF = 524288
TP_BN = 2048


def _tc_repack(table_t):
    nb = HALF // TP_BN
    vocab = table_t.shape[1]
    last_b = (vocab - 1) // TP_BN

    def body(xa_ref, xb_ref, o_ref):
        o_ref[:, 0:EMBED] = jnp.transpose(xa_ref[...], (1, 0))
        o_ref[:, EMBED:OUT_W] = jnp.transpose(xb_ref[...], (1, 0))

    return pl.pallas_call(
        body,
        grid=(nb,),
        in_specs=[
            pl.BlockSpec((EMBED, TP_BN), lambda c: (0, c)),
            pl.BlockSpec((EMBED, TP_BN),
                         lambda c: (0, jnp.minimum(c + nb, last_b))),
        ],
        out_specs=pl.BlockSpec((TP_BN, OUT_W), lambda c: (c, 0)),
        out_shape=jax.ShapeDtypeStruct((HALF, OUT_W), jnp.float32),
    )(table_t, table_t)


def _sc_gather(table, idx3, n_rows):
    chunks = idx3.shape[1]
    groups = chunks // DMAS_PER_GROUP
    rows_per_w = chunks * ROWS_PER_DMA

    mesh = plsc.VectorSubcoreMesh(core_axis_name="c", subcore_axis_name="s",
                                  num_cores=2, num_subcores=16)

    def body(table_hbm, idx_hbm, out_hbm, idx_v, buf0, buf1, gsem,
             wsem0, wsem1):
        bufs = (buf0, buf1)
        wsems = (wsem0, wsem1)
        cid = lax.axis_index("c")
        sid = lax.axis_index("s")
        wid = sid * 2 + cid
        base = wid * rows_per_w
        pltpu.sync_copy(idx_hbm.at[wid], idx_v)

        def wb_drain(b):
            pltpu.make_async_copy(
                bufs[b],
                out_hbm.at[pl.ds(0, GROUP_ROWS), pl.ds(0, EMBED)],
                wsems[b]).wait()

        @pl.loop(0, groups, step=2)
        def _(big_g):
            for b in range(2):
                g = big_g + b

                @pl.when(g >= 2)
                def _():
                    wb_drain(b)

                cps = []
                for k in range(DMAS_PER_GROUP):
                    j = g * DMAS_PER_GROUP + k
                    cps.append(pltpu.async_copy(
                        table_hbm.at[idx_v.at[j]],
                        bufs[b].at[pl.ds(k * ROWS_PER_DMA, ROWS_PER_DMA)],
                        gsem))
                for cp in cps:
                    cp.wait()
                pltpu.async_copy(
                    bufs[b],
                    out_hbm.at[pl.ds(base + g * GROUP_ROWS, GROUP_ROWS),
                               pl.ds(0, EMBED)],
                    wsems[b])

        for b in range(2):
            wb_drain(b)

    gathered = pl.kernel(
        body,
        out_type=jax.ShapeDtypeStruct((n_rows, OUT_W), jnp.float32),
        mesh=mesh,
        compiler_params=pltpu.CompilerParams(use_tc_tiling_on_sc=False),
        scratch_types=[
            pltpu.VMEM((chunks, ROWS_PER_DMA), jnp.int32),
            pltpu.VMEM((GROUP_ROWS, EMBED), jnp.float32),
            pltpu.VMEM((GROUP_ROWS, EMBED), jnp.float32),
            pltpu.SemaphoreType.DMA,
            pltpu.SemaphoreType.DMA,
            pltpu.SemaphoreType.DMA,
        ],
    )(table, idx3)
    return gathered


def _tc_linear_t(g3, w, b2, seq, batch):
    bb = 2048
    grid = (seq, batch // bb)

    def body(x_ref, w_ref, b_ref, o_ref):
        x = x_ref[0, :, 0:EMBED]
        o = lax.dot_general(w_ref[...], x, (((0,), (1,)), ((), ())),
                            preferred_element_type=jnp.float32)
        o_ref[0] = o + b_ref[...]

    return pl.pallas_call(
        body,
        grid=grid,
        in_specs=[
            pl.BlockSpec((1, bb, OUT_W), lambda s, c: (s, c, 0)),
            pl.BlockSpec((EMBED, EMBED), lambda s, c: (0, 0)),
            pl.BlockSpec((EMBED, 1), lambda s, c: (0, 0)),
        ],
        out_specs=pl.BlockSpec((1, EMBED, bb), lambda s, c: (s, 0, c)),
        out_shape=jax.ShapeDtypeStruct((seq, EMBED, batch), jnp.float32),
    )(g3, w, b2)


def kernel(qus, table, W, b):
    batch, seq = qus.shape
    n = batch * seq
    table_rm = _tc_repack(jnp.swapaxes(table, 0, 1)).reshape(2 * HALF, EMBED)
    idx = jnp.swapaxes(qus, 0, 1).reshape(n).astype(jnp.int32)
    idx = jnp.where(idx < HALF, 2 * idx, 2 * idx - (2 * HALF - 1))
    idx3 = idx.reshape(NW, n // (NW * ROWS_PER_DMA), ROWS_PER_DMA)
    gathered = _sc_gather(table_rm, idx3, n)
    g3 = gathered.reshape(seq, batch, OUT_W)
    out_t = _tc_linear_t(g3, W, b.reshape(EMBED, 1), seq, batch)
    return jnp.transpose(out_t, (2, 0, 1))


# speedup vs baseline: 2.4240x; 1.1963x over previous
import jax
import jax.numpy as jnp
from jax import lax
from jax.experimental import pallas as pl
from jax.experimental.pallas import tpu as pltpu
from jax.experimental.pallas import tpu_sc as plsc

EMBED = 64
HE = 32
NW = 32
ROWS_PER_DMA = 128
DMAS_PER_GROUP = 5
GROUP_ROWS = ROWS_PER_DMA * DMAS_PER_GROUP
OUT_W = 128
QUARTER = 262144
TP_BN = 4096


def _tc_repack(table_t):
    nbq = QUARTER // TP_BN
    vocab = table_t.shape[1]
    last_b = (vocab - 1) // TP_BN

    def pack(x_ref):
        lo = jnp.transpose(x_ref[0:HE, :], (1, 0))
        hi = jnp.transpose(x_ref[HE:EMBED, :], (1, 0))
        lo16 = lo.astype(jnp.bfloat16).astype(jnp.float32)
        hi16 = hi.astype(jnp.bfloat16).astype(jnp.float32)
        lo_i = lax.bitcast_convert_type(lo16, jnp.int32)
        hi_i = lax.bitcast_convert_type(hi16, jnp.int32)
        return lax.shift_right_logical(lo_i, 16) | (hi_i & jnp.int32(-65536))

    def body(xa_ref, xb_ref, xc_ref, xd_ref, o_ref):
        o_ref[:, 0 * HE:1 * HE] = pack(xa_ref)
        o_ref[:, 1 * HE:2 * HE] = pack(xb_ref)
        o_ref[:, 2 * HE:3 * HE] = pack(xc_ref)
        o_ref[:, 3 * HE:4 * HE] = pack(xd_ref)

    def spec(k):
        if k == 0:
            return pl.BlockSpec((EMBED, TP_BN), lambda c: (0, c))
        return pl.BlockSpec(
            (EMBED, TP_BN),
            lambda c, k=k: (0, jnp.minimum(c + k * nbq, last_b)))

    return pl.pallas_call(
        body,
        grid=(nbq,),
        in_specs=[spec(0), spec(1), spec(2), spec(3)],
        out_specs=pl.BlockSpec((TP_BN, OUT_W), lambda c: (c, 0)),
        out_shape=jax.ShapeDtypeStruct((QUARTER, OUT_W), jnp.int32),
    )(table_t, table_t, table_t, table_t)


def _sc_gather(table, idx3, n_rows):
    chunks = idx3.shape[1]
    groups = chunks // DMAS_PER_GROUP
    rows_per_w = chunks * ROWS_PER_DMA

    mesh = plsc.VectorSubcoreMesh(core_axis_name="c", subcore_axis_name="s",
                                  num_cores=2, num_subcores=16)

    def body(table_hbm, idx_hbm, out_hbm, idx_v, buf0, buf1, gsem,
             wsem0, wsem1):
        bufs = (buf0, buf1)
        wsems = (wsem0, wsem1)
        cid = lax.axis_index("c")
        sid = lax.axis_index("s")
        wid = sid * 2 + cid
        base = wid * rows_per_w
        pltpu.sync_copy(idx_hbm.at[wid], idx_v)

        def wb_drain(b):
            pltpu.make_async_copy(
                bufs[b],
                out_hbm.at[pl.ds(0, GROUP_ROWS), pl.ds(0, HE)],
                wsems[b]).wait()

        @pl.loop(0, groups, step=2)
        def _(big_g):
            for b in range(2):
                g = big_g + b

                @pl.when(g >= 2)
                def _():
                    wb_drain(b)

                cps = []
                for k in range(DMAS_PER_GROUP):
                    j = g * DMAS_PER_GROUP + k
                    cps.append(pltpu.async_copy(
                        table_hbm.at[idx_v.at[j]],
                        bufs[b].at[pl.ds(k * ROWS_PER_DMA, ROWS_PER_DMA)],
                        gsem))
                for cp in cps:
                    cp.wait()
                pltpu.async_copy(
                    bufs[b],
                    out_hbm.at[pl.ds(base + g * GROUP_ROWS, GROUP_ROWS),
                               pl.ds(0, HE)],
                    wsems[b])

        for b in range(2):
            wb_drain(b)

    gathered = pl.kernel(
        body,
        out_type=jax.ShapeDtypeStruct((n_rows, OUT_W), jnp.int32),
        mesh=mesh,
        compiler_params=pltpu.CompilerParams(use_tc_tiling_on_sc=False),
        scratch_types=[
            pltpu.VMEM((chunks, ROWS_PER_DMA), jnp.int32),
            pltpu.VMEM((GROUP_ROWS, HE), jnp.int32),
            pltpu.VMEM((GROUP_ROWS, HE), jnp.int32),
            pltpu.SemaphoreType.DMA,
            pltpu.SemaphoreType.DMA,
            pltpu.SemaphoreType.DMA,
        ],
    )(table, idx3)
    return gathered


def _tc_linear_t(g3, w_lo, w_hi, b2, seq, batch):
    bb = 2048
    grid = (seq, batch // bb)

    def body(x_ref, wl_ref, wh_ref, b_ref, o_ref):
        u = x_ref[0, :, 0:HE]
        xe = lax.bitcast_convert_type(lax.shift_left(u, 16), jnp.float32)
        xo = lax.bitcast_convert_type(u & jnp.int32(-65536), jnp.float32)
        o = (lax.dot_general(wl_ref[...], xe, (((0,), (1,)), ((), ())),
                             preferred_element_type=jnp.float32)
             + lax.dot_general(wh_ref[...], xo, (((0,), (1,)), ((), ())),
                               preferred_element_type=jnp.float32))
        o_ref[0] = o + b_ref[...]

    return pl.pallas_call(
        body,
        grid=grid,
        in_specs=[
            pl.BlockSpec((1, bb, OUT_W), lambda s, c: (s, c, 0)),
            pl.BlockSpec((HE, EMBED), lambda s, c: (0, 0)),
            pl.BlockSpec((HE, EMBED), lambda s, c: (0, 0)),
            pl.BlockSpec((EMBED, 1), lambda s, c: (0, 0)),
        ],
        out_specs=pl.BlockSpec((1, EMBED, bb), lambda s, c: (s, 0, c)),
        out_shape=jax.ShapeDtypeStruct((seq, EMBED, batch), jnp.float32),
    )(g3, w_lo, w_hi, b2)


def kernel(qus, table, W, b):
    batch, seq = qus.shape
    n = batch * seq
    table_rm = _tc_repack(jnp.swapaxes(table, 0, 1)).reshape(4 * QUARTER, HE)
    idx = jnp.swapaxes(qus, 0, 1).reshape(n).astype(jnp.int32)
    idx = ((idx & (QUARTER - 1)) << 2) | lax.shift_right_logical(idx, 18)
    idx3 = idx.reshape(NW, n // (NW * ROWS_PER_DMA), ROWS_PER_DMA)
    gathered = _sc_gather(table_rm, idx3, n)
    g3 = gathered.reshape(seq, batch, OUT_W)
    out_t = _tc_linear_t(g3, W[0:HE], W[HE:EMBED], b.reshape(EMBED, 1),
                         seq, batch)
    return jnp.transpose(out_t, (2, 0, 1))


# speedup vs baseline: 2.5013x; 1.0319x over previous
import jax
import jax.numpy as jnp
from jax import lax
from jax.experimental import pallas as pl
from jax.experimental.pallas import tpu as pltpu
from jax.experimental.pallas import tpu_sc as plsc

EMBED = 64
HE = 32
NW = 32
ROWS_PER_DMA = 128
DMAS_PER_GROUP = 5
GROUP_ROWS = ROWS_PER_DMA * DMAS_PER_GROUP
OUT_W = 128
QUARTER = 262144
TP_BN = 8192


def _tc_repack(table_t):
    nbq = QUARTER // TP_BN
    vocab = table_t.shape[1]
    last_b = (vocab - 1) // TP_BN

    def pack(x_ref):
        t = jnp.transpose(x_ref[...].astype(jnp.bfloat16), (1, 0))
        lo = lax.bitcast_convert_type(t[:, 0:HE], jnp.uint16)
        hi = lax.bitcast_convert_type(t[:, HE:EMBED], jnp.uint16)
        return lo.astype(jnp.int32) | lax.shift_left(hi.astype(jnp.int32), 16)

    def body(xa_ref, xb_ref, xc_ref, xd_ref, o_ref):
        o_ref[:, 0 * HE:1 * HE] = pack(xa_ref)
        o_ref[:, 1 * HE:2 * HE] = pack(xb_ref)
        o_ref[:, 2 * HE:3 * HE] = pack(xc_ref)
        o_ref[:, 3 * HE:4 * HE] = pack(xd_ref)

    def spec(k):
        if k == 0:
            return pl.BlockSpec((EMBED, TP_BN), lambda c: (0, c))
        return pl.BlockSpec(
            (EMBED, TP_BN),
            lambda c, k=k: (0, jnp.minimum(c + k * nbq, last_b)))

    return pl.pallas_call(
        body,
        grid=(nbq,),
        in_specs=[spec(0), spec(1), spec(2), spec(3)],
        out_specs=pl.BlockSpec((TP_BN, OUT_W), lambda c: (c, 0)),
        out_shape=jax.ShapeDtypeStruct((QUARTER, OUT_W), jnp.int32),
    )(table_t, table_t, table_t, table_t)


def _sc_gather(table, idx3, n_rows):
    chunks = idx3.shape[1]
    groups = chunks // DMAS_PER_GROUP
    rows_per_w = chunks * ROWS_PER_DMA

    mesh = plsc.VectorSubcoreMesh(core_axis_name="c", subcore_axis_name="s",
                                  num_cores=2, num_subcores=16)

    def body(table_hbm, idx_hbm, out_hbm, idx_v, buf0, buf1, gsem,
             wsem0, wsem1):
        bufs = (buf0, buf1)
        wsems = (wsem0, wsem1)
        cid = lax.axis_index("c")
        sid = lax.axis_index("s")
        wid = sid * 2 + cid
        base = wid * rows_per_w
        pltpu.sync_copy(idx_hbm.at[wid], idx_v)

        def wb_drain(b):
            pltpu.make_async_copy(
                bufs[b],
                out_hbm.at[pl.ds(0, GROUP_ROWS), pl.ds(0, HE)],
                wsems[b]).wait()

        @pl.loop(0, groups, step=2)
        def _(big_g):
            for b in range(2):
                g = big_g + b

                @pl.when(g >= 2)
                def _():
                    wb_drain(b)

                cps = []
                for k in range(DMAS_PER_GROUP):
                    j = g * DMAS_PER_GROUP + k
                    cps.append(pltpu.async_copy(
                        table_hbm.at[idx_v.at[j]],
                        bufs[b].at[pl.ds(k * ROWS_PER_DMA, ROWS_PER_DMA)],
                        gsem))
                for cp in cps:
                    cp.wait()
                pltpu.async_copy(
                    bufs[b],
                    out_hbm.at[pl.ds(base + g * GROUP_ROWS, GROUP_ROWS),
                               pl.ds(0, HE)],
                    wsems[b])

        for b in range(2):
            wb_drain(b)

    gathered = pl.kernel(
        body,
        out_type=jax.ShapeDtypeStruct((n_rows, OUT_W), jnp.int32),
        mesh=mesh,
        compiler_params=pltpu.CompilerParams(use_tc_tiling_on_sc=False),
        scratch_types=[
            pltpu.VMEM((chunks, ROWS_PER_DMA), jnp.int32),
            pltpu.VMEM((GROUP_ROWS, HE), jnp.int32),
            pltpu.VMEM((GROUP_ROWS, HE), jnp.int32),
            pltpu.SemaphoreType.DMA,
            pltpu.SemaphoreType.DMA,
            pltpu.SemaphoreType.DMA,
        ],
    )(table, idx3)
    return gathered


def _tc_linear_t(g3, w_lo, w_hi, b2, seq, batch):
    bb = 2048
    grid = (seq, batch // bb)

    def body(x_ref, wl_ref, wh_ref, b_ref, o_ref):
        u = x_ref[0, :, 0:HE]
        xe = lax.bitcast_convert_type(lax.shift_left(u, 16), jnp.float32)
        xo = lax.bitcast_convert_type(u & jnp.int32(-65536), jnp.float32)
        o = (lax.dot_general(wl_ref[...], xe, (((0,), (1,)), ((), ())),
                             preferred_element_type=jnp.float32)
             + lax.dot_general(wh_ref[...], xo, (((0,), (1,)), ((), ())),
                               preferred_element_type=jnp.float32))
        o_ref[0] = o + b_ref[...]

    return pl.pallas_call(
        body,
        grid=grid,
        in_specs=[
            pl.BlockSpec((1, bb, OUT_W), lambda s, c: (s, c, 0)),
            pl.BlockSpec((HE, EMBED), lambda s, c: (0, 0)),
            pl.BlockSpec((HE, EMBED), lambda s, c: (0, 0)),
            pl.BlockSpec((EMBED, 1), lambda s, c: (0, 0)),
        ],
        out_specs=pl.BlockSpec((1, EMBED, bb), lambda s, c: (s, 0, c)),
        out_shape=jax.ShapeDtypeStruct((seq, EMBED, batch), jnp.float32),
    )(g3, w_lo, w_hi, b2)


def kernel(qus, table, W, b):
    batch, seq = qus.shape
    n = batch * seq
    table_rm = _tc_repack(jnp.swapaxes(table, 0, 1)).reshape(4 * QUARTER, HE)
    idx = jnp.swapaxes(qus, 0, 1).reshape(n).astype(jnp.int32)
    idx = ((idx & (QUARTER - 1)) << 2) | lax.shift_right_logical(idx, 18)
    idx3 = idx.reshape(NW, n // (NW * ROWS_PER_DMA), ROWS_PER_DMA)
    gathered = _sc_gather(table_rm, idx3, n)
    g3 = gathered.reshape(seq, batch, OUT_W)
    out_t = _tc_linear_t(g3, W[0:HE], W[HE:EMBED], b.reshape(EMBED, 1),
                         seq, batch)
    return jnp.transpose(out_t, (2, 0, 1))


# speedup vs baseline: 2.6864x; 1.0740x over previous
import jax
import jax.numpy as jnp
from jax import lax
from jax.experimental import pallas as pl
from jax.experimental.pallas import tpu as pltpu
from jax.experimental.pallas import tpu_sc as plsc

EMBED = 64
HE = 32
NW = 32
ROWS_PER_DMA = 128
DMAS_PER_GROUP = 5
GROUP_ROWS = ROWS_PER_DMA * DMAS_PER_GROUP
OUT_W = 128
QUARTER = 262144
TP_BN = 8192


def _tc_repack(table_t):
    nbq = QUARTER // TP_BN
    vocab = table_t.shape[1]
    last_b = (vocab - 1) // TP_BN

    def pack(x_ref):
        t = jnp.transpose(x_ref[...].astype(jnp.bfloat16), (1, 0))
        lo = lax.bitcast_convert_type(t[:, 0:HE], jnp.uint16)
        hi = lax.bitcast_convert_type(t[:, HE:EMBED], jnp.uint16)
        return lo.astype(jnp.int32) | lax.shift_left(hi.astype(jnp.int32), 16)

    def body(xa_ref, xb_ref, xc_ref, xd_ref, o_ref):
        o_ref[:, 0 * HE:1 * HE] = pack(xa_ref)
        o_ref[:, 1 * HE:2 * HE] = pack(xb_ref)
        o_ref[:, 2 * HE:3 * HE] = pack(xc_ref)
        o_ref[:, 3 * HE:4 * HE] = pack(xd_ref)

    def spec(k):
        if k == 0:
            return pl.BlockSpec((EMBED, TP_BN), lambda c: (0, c))
        return pl.BlockSpec(
            (EMBED, TP_BN),
            lambda c, k=k: (0, jnp.minimum(c + k * nbq, last_b)))

    return pl.pallas_call(
        body,
        grid=(nbq,),
        in_specs=[spec(0), spec(1), spec(2), spec(3)],
        out_specs=pl.BlockSpec((TP_BN, OUT_W), lambda c: (c, 0)),
        out_shape=jax.ShapeDtypeStruct((QUARTER, OUT_W), jnp.int32),
    )(table_t, table_t, table_t, table_t)


def _sc_gather(table, idx3, n_rows):
    chunks = idx3.shape[1]
    groups = chunks // DMAS_PER_GROUP
    rows_per_w = chunks * ROWS_PER_DMA

    mesh = plsc.VectorSubcoreMesh(core_axis_name="c", subcore_axis_name="s",
                                  num_cores=2, num_subcores=16)

    def body(table_hbm, idx_hbm, out_hbm, idx_v, buf0, buf1, gsem,
             wsem0, wsem1):
        bufs = (buf0, buf1)
        wsems = (wsem0, wsem1)
        cid = lax.axis_index("c")
        sid = lax.axis_index("s")
        wid = sid * 2 + cid
        base = wid * rows_per_w
        pltpu.sync_copy(idx_hbm.at[wid], idx_v)

        def wb_drain(b):
            pltpu.make_async_copy(
                bufs[b],
                out_hbm.at[pl.ds(0, GROUP_ROWS), pl.ds(0, HE)],
                wsems[b]).wait()

        @pl.loop(0, groups, step=2)
        def _(big_g):
            for b in range(2):
                g = big_g + b

                @pl.when(g >= 2)
                def _():
                    wb_drain(b)

                cps = []
                for k in range(DMAS_PER_GROUP):
                    j = g * DMAS_PER_GROUP + k
                    cps.append(pltpu.async_copy(
                        table_hbm.at[idx_v.at[j]],
                        bufs[b].at[pl.ds(k * ROWS_PER_DMA, ROWS_PER_DMA)],
                        gsem))
                for cp in cps:
                    cp.wait()
                pltpu.async_copy(
                    bufs[b],
                    out_hbm.at[pl.ds(base + g * GROUP_ROWS, GROUP_ROWS),
                               pl.ds(0, HE)],
                    wsems[b])

        for b in range(2):
            wb_drain(b)

    gathered = pl.kernel(
        body,
        out_type=jax.ShapeDtypeStruct((n_rows, OUT_W), jnp.int32),
        mesh=mesh,
        compiler_params=pltpu.CompilerParams(use_tc_tiling_on_sc=False),
        scratch_types=[
            pltpu.VMEM((chunks, ROWS_PER_DMA), jnp.int32),
            pltpu.VMEM((GROUP_ROWS, HE), jnp.int32),
            pltpu.VMEM((GROUP_ROWS, HE), jnp.int32),
            pltpu.SemaphoreType.DMA,
            pltpu.SemaphoreType.DMA,
            pltpu.SemaphoreType.DMA,
        ],
    )(table, idx3)
    return gathered


def _tc_linear_t(g3, w_lo, w_hi, b2, seq, batch):
    bb = 4096
    grid = (seq, batch // bb)

    def body(x_ref, wl_ref, wh_ref, b_ref, o_ref):
        u = x_ref[0, :, 0:HE]
        xe = lax.bitcast_convert_type(lax.shift_left(u, 16), jnp.float32)
        xo = lax.bitcast_convert_type(u & jnp.int32(-65536), jnp.float32)
        o = (lax.dot_general(wl_ref[...], xe, (((0,), (1,)), ((), ())),
                             preferred_element_type=jnp.float32)
             + lax.dot_general(wh_ref[...], xo, (((0,), (1,)), ((), ())),
                               preferred_element_type=jnp.float32))
        o_ref[0] = o + b_ref[...]

    return pl.pallas_call(
        body,
        grid=grid,
        in_specs=[
            pl.BlockSpec((1, bb, OUT_W), lambda s, c: (s, c, 0)),
            pl.BlockSpec((HE, EMBED), lambda s, c: (0, 0)),
            pl.BlockSpec((HE, EMBED), lambda s, c: (0, 0)),
            pl.BlockSpec((EMBED, 1), lambda s, c: (0, 0)),
        ],
        out_specs=pl.BlockSpec((1, EMBED, bb), lambda s, c: (s, 0, c)),
        out_shape=jax.ShapeDtypeStruct((seq, EMBED, batch), jnp.float32),
    )(g3, w_lo, w_hi, b2)


def kernel(qus, table, W, b):
    batch, seq = qus.shape
    n = batch * seq
    table_rm = _tc_repack(jnp.swapaxes(table, 0, 1)).reshape(4 * QUARTER, HE)
    idx = jnp.swapaxes(qus, 0, 1).reshape(n).astype(jnp.int32)
    idx = ((idx & (QUARTER - 1)) << 2) | lax.shift_right_logical(idx, 18)
    idx3 = idx.reshape(NW, n // (NW * ROWS_PER_DMA), ROWS_PER_DMA)
    gathered = _sc_gather(table_rm, idx3, n)
    g3 = gathered.reshape(seq, batch, OUT_W)
    out_t = _tc_linear_t(g3, W[0:HE], W[HE:EMBED], b.reshape(EMBED, 1),
                         seq, batch)
    return jnp.transpose(out_t, (2, 0, 1))
